# baseline (speedup 1.0000x reference)
import functools

import jax
import jax.numpy as jnp
from jax import lax
from jax.experimental import pallas as pl
from jax.experimental.pallas import tpu as pltpu
from jax.experimental.pallas import tpu_sc as plsc

N = 10000
E = 320000
EL = 100000
D = 128
V = 65

TP0 = 112
T1 = 104
T2 = 96
T3 = 88
BN = 50

NC, NS = 2, 16
NW = NC * NS
NPAD = 10240
ROWS_PT = NPAD // NS
CE = 80
EPW = E // NW
ELP = 102400
ELPW = ELP // NW


def _cnn_body(s_ref, emb_ref, w1_ref, b1_ref, w2_ref, b2_ref, w3_ref,
              b3_ref, out_ref):
    bf = jnp.bfloat16
    s = s_ref[...]
    oh = (s == lax.broadcasted_iota(jnp.int32, (1, V), 1)).astype(bf)
    e2 = jnp.dot(oh, emb_ref[...], preferred_element_type=jnp.float32)
    e3 = e2.reshape(BN, TP0, D)

    acc1 = jnp.zeros((BN * T1, 40), jnp.float32)
    for k in range(4):
        ek = lax.slice(e3, (0, k, 0), (BN, k + T1, D)).reshape(BN * T1, D)
        acc1 += jnp.dot(ek.astype(bf), w1_ref[k * D:(k + 1) * D, :],
                        preferred_element_type=jnp.float32)
    h1 = jnp.maximum(acc1 + b1_ref[...], 0.0).reshape(BN, T1, 40)

    acc2 = jnp.zeros((BN * T2, 80), jnp.float32)
    for k in range(6):
        hk = lax.slice(h1, (0, k, 0), (BN, k + T2, 40)).reshape(BN * T2, 40)
        acc2 += jnp.dot(hk.astype(bf), w2_ref[k * 40:(k + 1) * 40, :],
                        preferred_element_type=jnp.float32)
    h2 = jnp.maximum(acc2 + b2_ref[...], 0.0).reshape(BN, T2, 80)

    acc3 = jnp.zeros((BN * T3, 160), jnp.float32)
    for k in range(8):
        hk = lax.slice(h2, (0, k, 0), (BN, k + T3, 80)).reshape(BN * T3, 80)
        acc3 += jnp.dot(hk.astype(bf), w3_ref[k * 80:(k + 1) * 80, :],
                        preferred_element_type=jnp.float32)
    y3 = jnp.maximum(acc3 + b3_ref[...], 0.0)
    out_ref[...] = y3.reshape(BN, T3, 160).astype(bf)


def _fc_body(y_ref, w_ref, b_ref, out_ref):
    @pl.when(pl.program_id(1) == 0)
    def _():
        out_ref[...] = jnp.broadcast_to(b_ref[...], out_ref.shape)

    out_ref[...] += jnp.dot(y_ref[...], w_ref[...],
                            preferred_element_type=jnp.float32)


def _combine1_body(p_ref, dp_ref, x_ref, wl_ref, wr_ref, b_ref, h_ref,
                   dinv_ref):
    s = p_ref[0] + p_ref[1]
    deg = dp_ref[0][:, 0:1] + dp_ref[1][:, 0:1]
    dinv = 1.0 / jnp.maximum(deg, 1.0)
    mean = s * dinv
    h = (jnp.dot(mean, wl_ref[...], preferred_element_type=jnp.float32)
         + jnp.dot(x_ref[...], wr_ref[...], preferred_element_type=jnp.float32)
         + b_ref[...])
    h_ref[...] = jnp.maximum(h, 0.0)
    dinv_ref[...] = dinv


def _combine2_body(p_ref, h_ref, dinv_ref, feat_ref, wl_ref, wr_ref, b_ref,
                   z_ref):
    s = p_ref[0] + p_ref[1]
    mean = s * dinv_ref[...]
    z_ref[...] = (jnp.dot(mean, wl_ref[...], preferred_element_type=jnp.float32)
                  + jnp.dot(h_ref[...], wr_ref[...],
                            preferred_element_type=jnp.float32)
                  + b_ref[...] + feat_ref[...])


def _dec_body(zr_ref, zc_ref, wt_ref, wb_ref, b1_ref, w2_ref, b2_ref, o_ref):
    h = (jnp.dot(zr_ref[...], wt_ref[...], preferred_element_type=jnp.float32)
         + jnp.dot(zc_ref[...], wb_ref[...], preferred_element_type=jnp.float32)
         + b1_ref[...])
    h = jnp.maximum(h, 0.0)
    o_ref[...] = jnp.sum(h * w2_ref[...], axis=1, keepdims=True) + b2_ref[...]


DEPTH = 5


def _make_segsum():
    zr = 16
    sdepth = 2
    nch = EPW // CE
    nset = nch // sdepth
    mesh = plsc.VectorSubcoreMesh(core_axis_name="c", subcore_axis_name="s",
                                  num_cores=NC, num_subcores=NS)

    @functools.partial(
        pl.kernel,
        out_type=jax.ShapeDtypeStruct((NC, NPAD, D), jnp.float32),
        mesh=mesh,
        scratch_types=[
            pltpu.VMEM(((nch + 1) // 2, 2, CE), jnp.int32),
            pltpu.VMEM((sdepth, CE, D), jnp.float32),
            pltpu.VMEM_SHARED((NPAD, D), jnp.float32),
            pltpu.SemaphoreType.DMA,
            pltpu.SemaphoreType.DMA,
        ],
    )
    def seg(table_h, sd_h, out_h, idxb, rows, acc, semg, sems):
        cid = lax.axis_index("c")
        sid = lax.axis_index("s")
        wid = sid * NC + cid
        zv = jnp.zeros((16,), jnp.float32)

        @pl.loop(0, CE)
        def _zb(r):
            for j in range(D // 16):
                rows[0, r, pl.ds(j * 16, 16)] = zv

        @pl.loop(0, ROWS_PT // CE)
        def _za(t):
            pltpu.sync_copy(rows.at[0],
                            acc.at[pl.ds(sid * ROWS_PT + t * CE, CE)])

        plsc.subcore_barrier()

        hlen = (nch + 1) // 2
        for ph, nc_ph in ((0, hlen), (1, nch - hlen)):
            pltpu.sync_copy(sd_h.at[pl.ds(wid * nch + ph * hlen, nc_ph)],
                            idxb.at[pl.ds(0, nc_ph)])

            @pl.loop(0, nc_ph // sdepth)
            def _set(j):
                c0 = j * sdepth
                gds = [pltpu.async_copy(table_h.at[idxb.at[c0 + p, 0]],
                                        rows.at[p], semg)
                       for p in range(sdepth)]
                for d in gds:
                    d.wait()
                sds = [pltpu.async_copy(rows.at[p],
                                        acc.at[idxb.at[c0 + p, 1]],
                                        sems, add=True)
                       for p in range(sdepth)]
                for d in sds:
                    d.wait()

            for c in range((nc_ph // sdepth) * sdepth, nc_ph):
                pltpu.async_copy(table_h.at[idxb.at[c, 0]], rows.at[0],
                                 semg).wait()
                pltpu.async_copy(rows.at[0], acc.at[idxb.at[c, 1]], sems,
                                 add=True).wait()

        plsc.subcore_barrier()
        pltpu.sync_copy(acc.at[pl.ds(sid * ROWS_PT, ROWS_PT)],
                        out_h.at[cid, pl.ds(sid * ROWS_PT, ROWS_PT)])

    return seg


def _make_deg():
    zr = 40
    mesh = plsc.VectorSubcoreMesh(core_axis_name="c", subcore_axis_name="s",
                                  num_cores=NC, num_subcores=NS)

    @functools.partial(
        pl.kernel,
        out_type=jax.ShapeDtypeStruct((NC, NPAD, D), jnp.float32),
        mesh=mesh,
        scratch_types=[
            pltpu.VMEM((EPW // CE, CE), jnp.int32),
            pltpu.VMEM((CE, D), jnp.float32),
            pltpu.VMEM((zr, D), jnp.float32),
            pltpu.VMEM_SHARED((NPAD, D), jnp.float32),
            pltpu.SemaphoreType.DMA,
        ],
    )
    def degk(dst_h, out_h, didx, ones_b, zbuf, acc, sems):
        cid = lax.axis_index("c")
        sid = lax.axis_index("s")
        wid = sid * NC + cid
        zv = jnp.zeros((16,), jnp.float32)
        ov = jnp.ones((16,), jnp.float32)
        nch = EPW // CE

        @pl.loop(0, zr)
        def _zb(r):
            for j in range(D // 16):
                zbuf[r, pl.ds(j * 16, 16)] = zv

        @pl.loop(0, CE)
        def _ob(r):
            for j in range(D // 16):
                ones_b[r, pl.ds(j * 16, 16)] = ov

        pltpu.sync_copy(dst_h.at[wid], didx)

        @pl.loop(0, ROWS_PT // zr)
        def _za(t):
            pltpu.sync_copy(zbuf, acc.at[pl.ds(sid * ROWS_PT + t * zr, zr)])

        plsc.subcore_barrier()

        @pl.loop(0, nch // DEPTH)
        def _set(j):
            c0 = j * DEPTH
            sds = [pltpu.async_copy(ones_b, acc.at[didx.at[c0 + p]],
                                    sems, add=True) for p in range(DEPTH)]
            for d in sds:
                d.wait()

        plsc.subcore_barrier()
        pltpu.sync_copy(acc.at[pl.ds(sid * ROWS_PT, ROWS_PT)],
                        out_h.at[cid, pl.ds(sid * ROWS_PT, ROWS_PT)])

    return degk


def _make_dec_gather():
    mesh = plsc.VectorSubcoreMesh(core_axis_name="c", subcore_axis_name="s", num_cores=NC, num_subcores=NS)
    sds = jax.ShapeDtypeStruct((ELP, D), jnp.float32)

    @functools.partial(
        pl.kernel,
        out_type=(sds, sds),
        mesh=mesh,
        scratch_types=[
            pltpu.VMEM((ELPW // CE, 2, CE), jnp.int32),
            pltpu.VMEM((DEPTH, CE, D), jnp.float32),
            pltpu.VMEM((DEPTH, CE, D), jnp.float32),
            pltpu.SemaphoreType.DMA,
            pltpu.SemaphoreType.DMA,
        ],
    )
    def dg(z_h, rc_h, zr_h, zc_h, idxb, rbuf, cbuf, semg, semw):
        cid = lax.axis_index("c")
        sid = lax.axis_index("s")
        wid = sid * NC + cid
        nch = ELPW // CE

        pltpu.sync_copy(rc_h.at[pl.ds(wid * nch, nch)], idxb)

        @pl.loop(0, nch // DEPTH)
        def _set(j):
            c0 = j * DEPTH
            gds = [pltpu.async_copy(z_h.at[idxb.at[c0 + p, 0]],
                                    rbuf.at[p], semg) for p in range(DEPTH)]
            gds += [pltpu.async_copy(z_h.at[idxb.at[c0 + p, 1]],
                                     cbuf.at[p], semg) for p in range(DEPTH)]
            for d in gds:
                d.wait()
            wds = []
            for p in range(DEPTH):
                base = pl.multiple_of((wid * nch + c0 + p) * CE, 8)
                wds.append(pltpu.async_copy(rbuf.at[p],
                                            zr_h.at[pl.ds(base, CE)], semw))
                wds.append(pltpu.async_copy(cbuf.at[p],
                                            zc_h.at[pl.ds(base, CE)], semw))
            for d in wds:
                d.wait()

    return dg


def kernel(x, edge_index, edge_label_index, smiles, emb,
           c1w, c1b, c2w, c2b, c3w, c3b, fcw, fcb,
           wl1, wr1, b1, wl2, wr2, b2, dw1, db1, dw2, db2):
    src = edge_index[0].astype(jnp.int32)
    dst = edge_index[1].astype(jnp.int32)
    row = jnp.pad(edge_label_index[0], (0, ELP - EL)).astype(jnp.int32)
    col = jnp.pad(edge_label_index[1], (0, ELP - EL)).astype(jnp.int32)

    sp = jnp.pad(smiles.astype(jnp.int32), ((0, 0), (0, TP0 - 100)),
                 constant_values=-1).reshape(N * TP0, 1)
    bf = jnp.bfloat16
    w1c = jnp.transpose(c1w, (2, 1, 0)).reshape(4 * D, 40).astype(bf)
    w2c = jnp.transpose(c2w, (2, 1, 0)).reshape(6 * 40, 80).astype(bf)
    w3c = jnp.transpose(c3w, (2, 1, 0)).reshape(8 * 80, 160).astype(bf)

    yconv = pl.pallas_call(
        _cnn_body,
        grid=(N // BN,),
        in_specs=[
            pl.BlockSpec((BN * TP0, 1), lambda i: (i, 0)),
            pl.BlockSpec((V, D), lambda i: (0, 0)),
            pl.BlockSpec((4 * D, 40), lambda i: (0, 0)),
            pl.BlockSpec((1, 40), lambda i: (0, 0)),
            pl.BlockSpec((240, 80), lambda i: (0, 0)),
            pl.BlockSpec((1, 80), lambda i: (0, 0)),
            pl.BlockSpec((640, 160), lambda i: (0, 0)),
            pl.BlockSpec((1, 160), lambda i: (0, 0)),
        ],
        out_specs=pl.BlockSpec((BN, T3, 160), lambda i: (i, 0, 0)),
        out_shape=jax.ShapeDtypeStruct((N, T3, 160), bf),
    )(sp, emb.astype(bf), w1c, c1b.reshape(1, 40), w2c, c2b.reshape(1, 80),
      w3c, c3b.reshape(1, 160))

    fcp = jnp.transpose(fcw.reshape(160, 85, D), (1, 0, 2))
    fcp = jnp.pad(fcp, ((0, T3 - 85), (0, 0), (0, 0)))
    fcp = fcp.reshape(T3 * 160, D).astype(bf)
    yflat = yconv.reshape(N, T3 * 160)
    kb = T3 * 160 // 10
    feat = pl.pallas_call(
        _fc_body,
        grid=(N // 400, 10),
        in_specs=[
            pl.BlockSpec((400, kb), lambda m, k: (m, k)),
            pl.BlockSpec((kb, D), lambda m, k: (k, 0)),
            pl.BlockSpec((1, D), lambda m, k: (0, 0)),
        ],
        out_specs=pl.BlockSpec((400, D), lambda m, k: (m, 0)),
        out_shape=jax.ShapeDtypeStruct((N, D), jnp.float32),
        compiler_params=pltpu.CompilerParams(
            dimension_semantics=("parallel", "arbitrary")),
    )(yflat, fcp, fcb.reshape(1, D))

    sd3 = jnp.stack([src.reshape(E // CE, CE), dst.reshape(E // CE, CE)],
                    axis=1)
    dst3 = dst.reshape(NW, EPW // CE, CE)
    part1 = _make_segsum()(x, sd3)
    degp = _make_deg()(dst3)

    h1, dinv = pl.pallas_call(
        _combine1_body,
        grid=(N // 400,),
        in_specs=[
            pl.BlockSpec((NC, 400, D), lambda i: (0, i, 0)),
            pl.BlockSpec((NC, 400, D), lambda i: (0, i, 0)),
            pl.BlockSpec((400, D), lambda i: (i, 0)),
            pl.BlockSpec((D, D), lambda i: (0, 0)),
            pl.BlockSpec((D, D), lambda i: (0, 0)),
            pl.BlockSpec((1, D), lambda i: (0, 0)),
        ],
        out_specs=[
            pl.BlockSpec((400, D), lambda i: (i, 0)),
            pl.BlockSpec((400, 1), lambda i: (i, 0)),
        ],
        out_shape=[
            jax.ShapeDtypeStruct((N, D), jnp.float32),
            jax.ShapeDtypeStruct((N, 1), jnp.float32),
        ],
    )(part1, degp, x, wl1, wr1, b1.reshape(1, D))

    part2 = _make_segsum()(h1, sd3)

    z = pl.pallas_call(
        _combine2_body,
        grid=(N // 400,),
        in_specs=[
            pl.BlockSpec((NC, 400, D), lambda i: (0, i, 0)),
            pl.BlockSpec((400, D), lambda i: (i, 0)),
            pl.BlockSpec((400, 1), lambda i: (i, 0)),
            pl.BlockSpec((400, D), lambda i: (i, 0)),
            pl.BlockSpec((D, D), lambda i: (0, 0)),
            pl.BlockSpec((D, D), lambda i: (0, 0)),
            pl.BlockSpec((1, D), lambda i: (0, 0)),
        ],
        out_specs=pl.BlockSpec((400, D), lambda i: (i, 0)),
        out_shape=jax.ShapeDtypeStruct((N, D), jnp.float32),
    )(part2, h1, dinv, feat, wl2, wr2, b2.reshape(1, D))

    rc3 = jnp.stack([row.reshape(ELP // CE, CE), col.reshape(ELP // CE, CE)],
                    axis=1)
    zr, zc = _make_dec_gather()(z, rc3)

    scores = pl.pallas_call(
        _dec_body,
        grid=(ELP // 512,),
        in_specs=[
            pl.BlockSpec((512, D), lambda i: (i, 0)),
            pl.BlockSpec((512, D), lambda i: (i, 0)),
            pl.BlockSpec((D, D), lambda i: (0, 0)),
            pl.BlockSpec((D, D), lambda i: (0, 0)),
            pl.BlockSpec((1, D), lambda i: (0, 0)),
            pl.BlockSpec((1, D), lambda i: (0, 0)),
            pl.BlockSpec((1, 1), lambda i: (0, 0)),
        ],
        out_specs=pl.BlockSpec((512, 1), lambda i: (i, 0)),
        out_shape=jax.ShapeDtypeStruct((ELP, 1), jnp.float32),
    )(zr, zc, dw1[:D], dw1[D:], db1.reshape(1, D), dw2.reshape(1, D),
      db2.reshape(1, 1))

    return (z, scores[:EL])


# speedup vs baseline: 1.4767x; 1.4767x over previous
import functools

import jax
import jax.numpy as jnp
from jax import lax
from jax.experimental import pallas as pl
from jax.experimental.pallas import tpu as pltpu
from jax.experimental.pallas import tpu_sc as plsc

N = 10000
E = 320000
EL = 100000
D = 128
V = 65

TP0 = 112
T1 = 104
T2 = 96
T3 = 88
BN = 50

NC, NS = 2, 16
NW = NC * NS
NPAD = 10240
ROWS_PT = NPAD // NS
CE = 80
EPW = E // NW
ELP = 102400
ELPW = ELP // NW


def _cnn_body(s_ref, emb_ref, w1_ref, b1_ref, w2_ref, b2_ref, w3_ref,
              b3_ref, out_ref):
    bf = jnp.bfloat16
    s = s_ref[...]
    oh = (s == lax.broadcasted_iota(jnp.int32, (1, V), 1)).astype(jnp.float32)
    e2 = jnp.dot(oh, emb_ref[...], preferred_element_type=jnp.float32)
    e3 = e2.reshape(BN, TP0, D)

    x1 = jnp.concatenate(
        [lax.slice(e3, (0, k, 0), (BN, k + T1, D)).reshape(BN * T1, D)
         for k in range(4)], axis=1)
    acc1 = jnp.dot(x1, w1_ref[...], preferred_element_type=jnp.float32)
    h1 = jnp.maximum(acc1 + b1_ref[...], 0.0).reshape(BN, T1, 40)

    x2 = jnp.concatenate(
        [lax.slice(h1, (0, k, 0), (BN, k + T2, 40)).reshape(BN * T2, 40)
         for k in range(6)], axis=1)
    acc2 = jnp.dot(x2, w2_ref[...], preferred_element_type=jnp.float32)
    h2 = jnp.maximum(acc2 + b2_ref[...], 0.0).reshape(BN, T2, 80)

    x3 = jnp.concatenate(
        [lax.slice(h2, (0, k, 0), (BN, k + T3, 80)).reshape(BN * T3, 80)
         for k in range(8)], axis=1)
    acc3 = jnp.dot(x3, w3_ref[...], preferred_element_type=jnp.float32)
    y3 = jnp.maximum(acc3 + b3_ref[...], 0.0)
    out_ref[...] = y3.reshape(BN, T3, 160).astype(bf)


def _fc_body(y_ref, w_ref, b_ref, out_ref):
    @pl.when(pl.program_id(1) == 0)
    def _():
        out_ref[...] = jnp.broadcast_to(b_ref[...], out_ref.shape)

    out_ref[...] += jnp.dot(y_ref[...], w_ref[...],
                            preferred_element_type=jnp.float32)


def _combine1_body(p_ref, dp_ref, x_ref, wl_ref, wr_ref, b_ref, h_ref,
                   dinv_ref):
    s = p_ref[0] + p_ref[1]
    deg = dp_ref[0][:, 0:1] + dp_ref[1][:, 0:1]
    dinv = 1.0 / jnp.maximum(deg, 1.0)
    mean = s * dinv
    h = (jnp.dot(mean, wl_ref[...], preferred_element_type=jnp.float32)
         + jnp.dot(x_ref[...], wr_ref[...], preferred_element_type=jnp.float32)
         + b_ref[...])
    h_ref[...] = jnp.maximum(h, 0.0)
    dinv_ref[...] = dinv


def _combine2_body(p_ref, h_ref, dinv_ref, feat_ref, wl_ref, wr_ref, b_ref,
                   wt_ref, wb_ref, z_ref, u_ref, v_ref):
    s = p_ref[0] + p_ref[1]
    mean = s * dinv_ref[...]
    z = (jnp.dot(mean, wl_ref[...], preferred_element_type=jnp.float32)
         + jnp.dot(h_ref[...], wr_ref[...], preferred_element_type=jnp.float32)
         + b_ref[...] + feat_ref[...])
    z_ref[...] = z
    u_ref[...] = jnp.dot(z, wt_ref[...], preferred_element_type=jnp.float32)
    v_ref[...] = jnp.dot(z, wb_ref[...], preferred_element_type=jnp.float32)


def _dec_body(ur_ref, vc_ref, b1_ref, w2_ref, b2_ref, o_ref):
    h = jnp.maximum(ur_ref[...] + vc_ref[...] + b1_ref[...], 0.0)
    o_ref[...] = jnp.sum(h * w2_ref[...], axis=1, keepdims=True) + b2_ref[...]


DEPTH = 5


def _make_segsum():
    zr = 16
    sdepth = 2
    nch = EPW // CE
    nset = nch // sdepth
    mesh = plsc.VectorSubcoreMesh(core_axis_name="c", subcore_axis_name="s",
                                  num_cores=NC, num_subcores=NS)

    @functools.partial(
        pl.kernel,
        out_type=jax.ShapeDtypeStruct((NC, NPAD, D), jnp.float32),
        mesh=mesh,
        scratch_types=[
            pltpu.VMEM(((nch + 1) // 2, 2, CE), jnp.int32),
            pltpu.VMEM((sdepth, CE, D), jnp.float32),
            pltpu.VMEM_SHARED((NPAD, D), jnp.float32),
            pltpu.SemaphoreType.DMA,
            pltpu.SemaphoreType.DMA,
        ],
    )
    def seg(table_h, sd_h, out_h, idxb, rows, acc, semg, sems):
        cid = lax.axis_index("c")
        sid = lax.axis_index("s")
        wid = sid * NC + cid
        zv = jnp.zeros((16,), jnp.float32)

        @pl.loop(0, CE)
        def _zb(r):
            for j in range(D // 16):
                rows[0, r, pl.ds(j * 16, 16)] = zv

        @pl.loop(0, ROWS_PT // CE)
        def _za(t):
            pltpu.sync_copy(rows.at[0],
                            acc.at[pl.ds(sid * ROWS_PT + t * CE, CE)])

        plsc.subcore_barrier()

        hlen = (nch + 1) // 2
        for ph, nc_ph in ((0, hlen), (1, nch - hlen)):
            pltpu.sync_copy(sd_h.at[pl.ds(wid * nch + ph * hlen, nc_ph)],
                            idxb.at[pl.ds(0, nc_ph)])

            @pl.loop(0, nc_ph // sdepth)
            def _set(j):
                c0 = j * sdepth
                gds = [pltpu.async_copy(table_h.at[idxb.at[c0 + p, 0]],
                                        rows.at[p], semg)
                       for p in range(sdepth)]
                for d in gds:
                    d.wait()
                sds = [pltpu.async_copy(rows.at[p],
                                        acc.at[idxb.at[c0 + p, 1]],
                                        sems, add=True)
                       for p in range(sdepth)]
                for d in sds:
                    d.wait()

            for c in range((nc_ph // sdepth) * sdepth, nc_ph):
                pltpu.async_copy(table_h.at[idxb.at[c, 0]], rows.at[0],
                                 semg).wait()
                pltpu.async_copy(rows.at[0], acc.at[idxb.at[c, 1]], sems,
                                 add=True).wait()

        plsc.subcore_barrier()
        pltpu.sync_copy(acc.at[pl.ds(sid * ROWS_PT, ROWS_PT)],
                        out_h.at[cid, pl.ds(sid * ROWS_PT, ROWS_PT)])

    return seg


def _make_deg():
    zr = 40
    mesh = plsc.VectorSubcoreMesh(core_axis_name="c", subcore_axis_name="s",
                                  num_cores=NC, num_subcores=NS)

    @functools.partial(
        pl.kernel,
        out_type=jax.ShapeDtypeStruct((NC, NPAD, D), jnp.float32),
        mesh=mesh,
        scratch_types=[
            pltpu.VMEM((EPW // CE, CE), jnp.int32),
            pltpu.VMEM((CE, D), jnp.float32),
            pltpu.VMEM((zr, D), jnp.float32),
            pltpu.VMEM_SHARED((NPAD, D), jnp.float32),
            pltpu.SemaphoreType.DMA,
        ],
    )
    def degk(dst_h, out_h, didx, ones_b, zbuf, acc, sems):
        cid = lax.axis_index("c")
        sid = lax.axis_index("s")
        wid = sid * NC + cid
        zv = jnp.zeros((16,), jnp.float32)
        ov = jnp.ones((16,), jnp.float32)
        nch = EPW // CE

        @pl.loop(0, zr)
        def _zb(r):
            for j in range(D // 16):
                zbuf[r, pl.ds(j * 16, 16)] = zv

        @pl.loop(0, CE)
        def _ob(r):
            for j in range(D // 16):
                ones_b[r, pl.ds(j * 16, 16)] = ov

        pltpu.sync_copy(dst_h.at[wid], didx)

        @pl.loop(0, ROWS_PT // zr)
        def _za(t):
            pltpu.sync_copy(zbuf, acc.at[pl.ds(sid * ROWS_PT + t * zr, zr)])

        plsc.subcore_barrier()

        @pl.loop(0, nch // DEPTH)
        def _set(j):
            c0 = j * DEPTH
            sds = [pltpu.async_copy(ones_b, acc.at[didx.at[c0 + p]],
                                    sems, add=True) for p in range(DEPTH)]
            for d in sds:
                d.wait()

        plsc.subcore_barrier()
        pltpu.sync_copy(acc.at[pl.ds(sid * ROWS_PT, ROWS_PT)],
                        out_h.at[cid, pl.ds(sid * ROWS_PT, ROWS_PT)])

    return degk


def _make_dec_gather():
    mesh = plsc.VectorSubcoreMesh(core_axis_name="c", subcore_axis_name="s", num_cores=NC, num_subcores=NS)
    sds = jax.ShapeDtypeStruct((ELP, D), jnp.float32)

    @functools.partial(
        pl.kernel,
        out_type=(sds, sds),
        mesh=mesh,
        scratch_types=[
            pltpu.VMEM((ELPW // CE, 2, CE), jnp.int32),
            pltpu.VMEM((DEPTH, CE, D), jnp.float32),
            pltpu.VMEM((DEPTH, CE, D), jnp.float32),
            pltpu.SemaphoreType.DMA,
            pltpu.SemaphoreType.DMA,
        ],
    )
    def dg(u_h, v_h, rc_h, zr_h, zc_h, idxb, rbuf, cbuf, semg, semw):
        cid = lax.axis_index("c")
        sid = lax.axis_index("s")
        wid = sid * NC + cid
        nch = ELPW // CE

        pltpu.sync_copy(rc_h.at[pl.ds(wid * nch, nch)], idxb)

        @pl.loop(0, nch // DEPTH)
        def _set(j):
            c0 = j * DEPTH
            gds = [pltpu.async_copy(u_h.at[idxb.at[c0 + p, 0]],
                                    rbuf.at[p], semg) for p in range(DEPTH)]
            gds += [pltpu.async_copy(v_h.at[idxb.at[c0 + p, 1]],
                                     cbuf.at[p], semg) for p in range(DEPTH)]
            for d in gds:
                d.wait()
            wds = []
            for p in range(DEPTH):
                base = pl.multiple_of((wid * nch + c0 + p) * CE, 8)
                wds.append(pltpu.async_copy(rbuf.at[p],
                                            zr_h.at[pl.ds(base, CE)], semw))
                wds.append(pltpu.async_copy(cbuf.at[p],
                                            zc_h.at[pl.ds(base, CE)], semw))
            for d in wds:
                d.wait()

    return dg


def kernel(x, edge_index, edge_label_index, smiles, emb,
           c1w, c1b, c2w, c2b, c3w, c3b, fcw, fcb,
           wl1, wr1, b1, wl2, wr2, b2, dw1, db1, dw2, db2):
    src = edge_index[0].astype(jnp.int32)
    dst = edge_index[1].astype(jnp.int32)
    row = jnp.pad(edge_label_index[0], (0, ELP - EL)).astype(jnp.int32)
    col = jnp.pad(edge_label_index[1], (0, ELP - EL)).astype(jnp.int32)

    sp = jnp.pad(smiles.astype(jnp.int32), ((0, 0), (0, TP0 - 100)),
                 constant_values=-1).reshape(N * TP0, 1)
    bf = jnp.bfloat16
    w1c = jnp.transpose(c1w, (2, 1, 0)).reshape(4 * D, 40)
    w2c = jnp.transpose(c2w, (2, 1, 0)).reshape(6 * 40, 80)
    w3c = jnp.transpose(c3w, (2, 1, 0)).reshape(8 * 80, 160)

    yconv = pl.pallas_call(
        _cnn_body,
        grid=(N // BN,),
        in_specs=[
            pl.BlockSpec((BN * TP0, 1), lambda i: (i, 0)),
            pl.BlockSpec((V, D), lambda i: (0, 0)),
            pl.BlockSpec((4 * D, 40), lambda i: (0, 0)),
            pl.BlockSpec((1, 40), lambda i: (0, 0)),
            pl.BlockSpec((240, 80), lambda i: (0, 0)),
            pl.BlockSpec((1, 80), lambda i: (0, 0)),
            pl.BlockSpec((640, 160), lambda i: (0, 0)),
            pl.BlockSpec((1, 160), lambda i: (0, 0)),
        ],
        out_specs=pl.BlockSpec((BN, T3, 160), lambda i: (i, 0, 0)),
        out_shape=jax.ShapeDtypeStruct((N, T3, 160), bf),
    )(sp, emb, w1c, c1b.reshape(1, 40), w2c, c2b.reshape(1, 80),
      w3c, c3b.reshape(1, 160))

    fcp = jnp.transpose(fcw.reshape(160, 85, D), (1, 0, 2))
    fcp = jnp.pad(fcp, ((0, T3 - 85), (0, 0), (0, 0)))
    fcp = fcp.reshape(T3 * 160, D).astype(bf)
    yflat = yconv.reshape(N, T3 * 160)
    kb = T3 * 160 // 10
    feat = pl.pallas_call(
        _fc_body,
        grid=(N // 400, 10),
        in_specs=[
            pl.BlockSpec((400, kb), lambda m, k: (m, k)),
            pl.BlockSpec((kb, D), lambda m, k: (k, 0)),
            pl.BlockSpec((1, D), lambda m, k: (0, 0)),
        ],
        out_specs=pl.BlockSpec((400, D), lambda m, k: (m, 0)),
        out_shape=jax.ShapeDtypeStruct((N, D), jnp.float32),
        compiler_params=pltpu.CompilerParams(
            dimension_semantics=("parallel", "arbitrary")),
    )(yflat, fcp, fcb.reshape(1, D))

    sd3 = jnp.stack([src.reshape(E // CE, CE), dst.reshape(E // CE, CE)],
                    axis=1)
    dst3 = dst.reshape(NW, EPW // CE, CE)
    part1 = _make_segsum()(x, sd3)
    degp = _make_deg()(dst3)

    h1, dinv = pl.pallas_call(
        _combine1_body,
        grid=(N // 400,),
        in_specs=[
            pl.BlockSpec((NC, 400, D), lambda i: (0, i, 0)),
            pl.BlockSpec((NC, 400, D), lambda i: (0, i, 0)),
            pl.BlockSpec((400, D), lambda i: (i, 0)),
            pl.BlockSpec((D, D), lambda i: (0, 0)),
            pl.BlockSpec((D, D), lambda i: (0, 0)),
            pl.BlockSpec((1, D), lambda i: (0, 0)),
        ],
        out_specs=[
            pl.BlockSpec((400, D), lambda i: (i, 0)),
            pl.BlockSpec((400, 1), lambda i: (i, 0)),
        ],
        out_shape=[
            jax.ShapeDtypeStruct((N, D), jnp.float32),
            jax.ShapeDtypeStruct((N, 1), jnp.float32),
        ],
    )(part1, degp, x, wl1, wr1, b1.reshape(1, D))

    part2 = _make_segsum()(h1, sd3)

    z, u, v = pl.pallas_call(
        _combine2_body,
        grid=(N // 400,),
        in_specs=[
            pl.BlockSpec((NC, 400, D), lambda i: (0, i, 0)),
            pl.BlockSpec((400, D), lambda i: (i, 0)),
            pl.BlockSpec((400, 1), lambda i: (i, 0)),
            pl.BlockSpec((400, D), lambda i: (i, 0)),
            pl.BlockSpec((D, D), lambda i: (0, 0)),
            pl.BlockSpec((D, D), lambda i: (0, 0)),
            pl.BlockSpec((1, D), lambda i: (0, 0)),
            pl.BlockSpec((D, D), lambda i: (0, 0)),
            pl.BlockSpec((D, D), lambda i: (0, 0)),
        ],
        out_specs=[
            pl.BlockSpec((400, D), lambda i: (i, 0)),
            pl.BlockSpec((400, D), lambda i: (i, 0)),
            pl.BlockSpec((400, D), lambda i: (i, 0)),
        ],
        out_shape=[
            jax.ShapeDtypeStruct((N, D), jnp.float32),
            jax.ShapeDtypeStruct((N, D), jnp.float32),
            jax.ShapeDtypeStruct((N, D), jnp.float32),
        ],
    )(part2, h1, dinv, feat, wl2, wr2, b2.reshape(1, D), dw1[:D], dw1[D:])

    rc3 = jnp.stack([row.reshape(ELP // CE, CE), col.reshape(ELP // CE, CE)],
                    axis=1)
    ur, vc = _make_dec_gather()(u, v, rc3)

    scores = pl.pallas_call(
        _dec_body,
        grid=(ELP // 512,),
        in_specs=[
            pl.BlockSpec((512, D), lambda i: (i, 0)),
            pl.BlockSpec((512, D), lambda i: (i, 0)),
            pl.BlockSpec((1, D), lambda i: (0, 0)),
            pl.BlockSpec((1, D), lambda i: (0, 0)),
            pl.BlockSpec((1, 1), lambda i: (0, 0)),
        ],
        out_specs=pl.BlockSpec((512, 1), lambda i: (i, 0)),
        out_shape=jax.ShapeDtypeStruct((ELP, 1), jnp.float32),
    )(ur, vc, db1.reshape(1, D), dw2.reshape(1, D), db2.reshape(1, 1))

    return (z, scores[:EL])
